# baseline (speedup 1.0000x reference)
import functools

import jax
import jax.numpy as jnp
from jax import lax
from jax.experimental import pallas as pl
from jax.experimental.pallas import tpu as pltpu
from jax.experimental.pallas import tpu_sc as plsc

BATCH = 16384
EMBD = 64
OUT_PAD = 8

_info = plsc.get_sparse_core_info()
_NC = _info.num_cores
_NS = _info.num_subcores
_NW = _NC * _NS
_BPW = BATCH // _NW


def _sc_gather_body(users_hbm, movies_hbm, utab_hbm, mtab_hbm,
                    u_out_hbm, m_out_hbm,
                    uidx_v, midx_v, u_rows_v, m_rows_v, sem_u, sem_m):
    wid = lax.axis_index("s") * _NC + lax.axis_index("c")
    base = wid * _BPW
    pltpu.sync_copy(users_hbm.at[pl.ds(base, _BPW)], uidx_v)
    pltpu.sync_copy(movies_hbm.at[pl.ds(base, _BPW)], midx_v)
    cp_u = pltpu.async_copy(utab_hbm.at[uidx_v], u_rows_v, sem_u)
    cp_m = pltpu.async_copy(mtab_hbm.at[midx_v], m_rows_v, sem_m)
    cp_u.wait()
    cp_m.wait()
    pltpu.sync_copy(u_rows_v, u_out_hbm.at[pl.ds(base, _BPW)])
    pltpu.sync_copy(m_rows_v, m_out_hbm.at[pl.ds(base, _BPW)])


def _sc_gather(users, movies, user_table, movie_table):
    mesh = plsc.VectorSubcoreMesh(core_axis_name="c", subcore_axis_name="s")
    k = pl.kernel(
        _sc_gather_body,
        mesh=mesh,
        compiler_params=pltpu.CompilerParams(use_tc_tiling_on_sc=False),
        out_type=[
            jax.ShapeDtypeStruct((BATCH, EMBD), jnp.float32),
            jax.ShapeDtypeStruct((BATCH, EMBD), jnp.float32),
        ],
        scratch_types=[
            pltpu.VMEM((_BPW,), jnp.int32),
            pltpu.VMEM((_BPW,), jnp.int32),
            pltpu.VMEM((_BPW, EMBD), jnp.float32),
            pltpu.VMEM((_BPW, EMBD), jnp.float32),
            pltpu.SemaphoreType.DMA,
            pltpu.SemaphoreType.DMA,
        ],
    )
    return k(users, movies, user_table, movie_table)


def _tc_head_body(u_ref, m_ref, wu_ref, wm_ref, b_ref, out_ref):
    acc = jnp.dot(u_ref[...], wu_ref[...], preferred_element_type=jnp.float32)
    acc += jnp.dot(m_ref[...], wm_ref[...], preferred_element_type=jnp.float32)
    acc += b_ref[...]
    mx = jnp.max(acc, axis=-1, keepdims=True)
    e = jnp.exp(acc - mx)
    s = jnp.sum(e, axis=-1, keepdims=True)
    out_ref[...] = e / s


def _tc_head(u_rows, m_rows, wu, wm, b8):
    blk = 2048
    grid = BATCH // blk
    return pl.pallas_call(
        _tc_head_body,
        grid=(grid,),
        in_specs=[
            pl.BlockSpec((blk, EMBD), lambda i: (i, 0)),
            pl.BlockSpec((blk, EMBD), lambda i: (i, 0)),
            pl.BlockSpec((EMBD, OUT_PAD), lambda i: (0, 0)),
            pl.BlockSpec((EMBD, OUT_PAD), lambda i: (0, 0)),
            pl.BlockSpec((1, OUT_PAD), lambda i: (0, 0)),
        ],
        out_specs=pl.BlockSpec((blk, OUT_PAD), lambda i: (i, 0)),
        out_shape=jax.ShapeDtypeStruct((BATCH, OUT_PAD), jnp.float32),
    )(u_rows, m_rows, wu, wm, b8)


def kernel(users, movies, user_table, movie_table, W, b):
    u_rows, m_rows = _sc_gather(users, movies, user_table, movie_table)
    wu = jnp.pad(W[:EMBD], ((0, 0), (0, OUT_PAD - 5)))
    wm = jnp.pad(W[EMBD:], ((0, 0), (0, OUT_PAD - 5)))
    b8 = jnp.pad(b, (0, OUT_PAD - 5), constant_values=-1e30).reshape(1, OUT_PAD)
    out8 = _tc_head(u_rows, m_rows, wu, wm, b8)
    return out8[:, :5]


# speedup vs baseline: 1.5584x; 1.5584x over previous
import functools

import jax
import jax.numpy as jnp
from jax import lax
from jax.experimental import pallas as pl
from jax.experimental.pallas import tpu as pltpu
from jax.experimental.pallas import tpu_sc as plsc

BATCH = 16384
EMBD = 64
OUT_PAD = 8
CHUNK = 8
WIN = 32

_info = plsc.get_sparse_core_info()
_NC = _info.num_cores
_NS = _info.num_subcores
_NW = _NC * _NS
_BPW = BATCH // _NW
_NWIN = _BPW // WIN


def _gather_one_table(tab_hbm, out_hbm, base, idx_v, rowbuf_v, sem):
    def win_body(w, _):
        for g in range(WIN // 16):
            vec = idx_v[pl.ds(w * WIN + g * 16, 16)]
            for l in range(16):
                j = g * 16 + l
                pltpu.async_copy(tab_hbm.at[pl.ds(vec[l], 1)],
                                 rowbuf_v.at[pl.ds(j, 1)], sem)
        pltpu.make_async_copy(tab_hbm.at[pl.ds(0, WIN)], rowbuf_v, sem).wait()
        pltpu.sync_copy(rowbuf_v, out_hbm.at[pl.ds(base + w * WIN, WIN)])
        return _
    lax.fori_loop(0, _BPW // WIN, win_body, 0)


def _sc_gather_body(users_hbm, movies_hbm, utab_hbm, mtab_hbm,
                    u_out_hbm, m_out_hbm,
                    uidx_v, midx_v, rowbuf_v, sem):
    wid = lax.axis_index("s") * _NC + lax.axis_index("c")
    base = wid * _BPW
    pltpu.sync_copy(users_hbm.at[pl.ds(base, _BPW)], uidx_v)
    pltpu.sync_copy(movies_hbm.at[pl.ds(base, _BPW)], midx_v)
    _gather_one_table(utab_hbm, u_out_hbm, base, uidx_v, rowbuf_v, sem)
    _gather_one_table(mtab_hbm, m_out_hbm, base, midx_v, rowbuf_v, sem)


def _sc_gather(users, movies, utab, mtab):
    mesh = plsc.VectorSubcoreMesh(core_axis_name="c", subcore_axis_name="s")
    k = pl.kernel(
        _sc_gather_body,
        mesh=mesh,
        out_type=[
            jax.ShapeDtypeStruct((BATCH, EMBD), jnp.float32),
            jax.ShapeDtypeStruct((BATCH, EMBD), jnp.float32),
        ],
        scratch_types=[
            pltpu.VMEM((_BPW,), jnp.int32),
            pltpu.VMEM((_BPW,), jnp.int32),
            pltpu.VMEM((WIN, EMBD), jnp.float32),
            pltpu.SemaphoreType.DMA,
        ],
    )
    return k(users, movies, utab, mtab)


def _tc_head_body(u_ref, m_ref, wu_ref, wm_ref, b_ref, out_ref):
    acc = jnp.dot(u_ref[...], wu_ref[...], preferred_element_type=jnp.float32)
    acc += jnp.dot(m_ref[...], wm_ref[...], preferred_element_type=jnp.float32)
    acc += b_ref[...]
    mx = jnp.max(acc, axis=-1, keepdims=True)
    e = jnp.exp(acc - mx)
    s = jnp.sum(e, axis=-1, keepdims=True)
    out_ref[...] = e / s


def _tc_head(u_rows, m_rows, wu, wm, b8):
    blk = 2048
    grid = BATCH // blk
    return pl.pallas_call(
        _tc_head_body,
        grid=(grid,),
        in_specs=[
            pl.BlockSpec((blk, EMBD), lambda i: (i, 0)),
            pl.BlockSpec((blk, EMBD), lambda i: (i, 0)),
            pl.BlockSpec((EMBD, OUT_PAD), lambda i: (0, 0)),
            pl.BlockSpec((EMBD, OUT_PAD), lambda i: (0, 0)),
            pl.BlockSpec((1, OUT_PAD), lambda i: (0, 0)),
        ],
        out_specs=pl.BlockSpec((blk, OUT_PAD), lambda i: (i, 0)),
        out_shape=jax.ShapeDtypeStruct((BATCH, OUT_PAD), jnp.float32),
    )(u_rows, m_rows, wu, wm, b8)


def kernel(users, movies, user_table, movie_table, W, b):
    u_rows, m_rows = _sc_gather(users, movies, user_table, movie_table)
    wu = jnp.pad(W[:EMBD], ((0, 0), (0, OUT_PAD - 5)))
    wm = jnp.pad(W[EMBD:], ((0, 0), (0, OUT_PAD - 5)))
    b8 = jnp.pad(b, (0, OUT_PAD - 5), constant_values=-1e30).reshape(1, OUT_PAD)
    out8 = _tc_head(u_rows, m_rows, wu, wm, b8)
    return out8[:, :5]


# speedup vs baseline: 1.8900x; 1.2128x over previous
import functools

import jax
import jax.numpy as jnp
from jax import lax
from jax.experimental import pallas as pl
from jax.experimental.pallas import tpu as pltpu
from jax.experimental.pallas import tpu_sc as plsc

BATCH = 16384
EMBD = 64
OUT_PAD = 8
WLANES = 256
RING = 8

_info = plsc.get_sparse_core_info()
_NC = _info.num_cores
_NS = _info.num_subcores
_NW = _NC * _NS

N_U = 1000000
N_M = 100000
_U_TAIL = [(999936, 128)]
_M_TAIL = [(99840, 128), (99968, 128)]

_IOTA = None


def _phase(tab_hbm, stage_hbm, idx_v, mpos_v, mlane_v, wpos_v, wlane_v,
           buf_v, rowtmp_v, sem_s, sem_w, wid, n_total, tails):
    iota = lax.iota(jnp.int32, 16)
    shard = n_total // _NW
    lo = lax.bitwise_and(shard * wid, ~(WLANES - 1))
    hi_main = lax.bitwise_and(shard * (wid + 1), ~(WLANES - 1))
    hi = jnp.where(wid == _NW - 1, n_total, hi_main)

    def scan_body(g, c):
        v = idx_v[pl.ds(g * 16, 16)]
        inr = jnp.logical_and(v >= lo, v < hi)
        plsc.store_compressed(mpos_v.at[pl.ds(c, 16)], iota + g * 16, mask=inr)
        plsc.store_compressed(mlane_v.at[pl.ds(c, 16)], v, mask=inr)
        return c + plsc.all_reduce_population_count(inr)[0]
    n = lax.fori_loop(0, BATCH // 16, scan_body, 0)

    def do_window(wlo, wsize, cw):
        wlo = pl.multiple_of(wlo, 128)
        pltpu.async_copy(tab_hbm.at[:, pl.ds(wlo, wsize)],
                         buf_v.at[:, pl.ds(0, wsize)], sem_s).wait()
        def act_body(g, c2):
            mv = mlane_v[pl.ds(g * 16, 16)]
            pv = mpos_v[pl.ds(g * 16, 16)]
            a = jnp.logical_and(mv >= wlo, mv < wlo + wsize)
            a = jnp.logical_and(a, iota < n - g * 16)
            plsc.store_compressed(wlane_v.at[pl.ds(c2, 16)], mv, mask=a)
            plsc.store_compressed(wpos_v.at[pl.ds(c2, 16)], pv, mask=a)
            return c2 + plsc.all_reduce_population_count(a)[0]
        na = lax.fori_loop(0, (n + 15) // 16, act_body, 0)

        def ext_body(i, c3):
            pv = wpos_v[pl.ds(i, 16)]
            lv = wlane_v[pl.ds(i, 16)]
            pos = pv[0]
            ll = lv[0] - wlo
            llv = jnp.broadcast_to(ll, (16,))
            slot = lax.bitwise_and(c3, RING - 1)
            for g4 in range(4):
                col = plsc.load_gather(buf_v, [iota + g4 * 16, llv])
                rowtmp_v[slot, pl.ds(g4 * 16, 16)] = col

            @pl.when(c3 >= RING)
            def _():
                pltpu.make_async_copy(stage_hbm.at[pl.ds(0, 1)],
                                      rowtmp_v.at[pl.ds(0, 1)], sem_w).wait()
            pltpu.async_copy(rowtmp_v.at[pl.ds(slot, 1)],
                             stage_hbm.at[pl.ds(pos, 1)], sem_w)
            return c3 + 1
        return lax.fori_loop(0, na, ext_body, cw)

    nw = lax.shift_right_logical(hi_main - lo, 8)

    def win_body(w, cw):
        return do_window(lo + w * WLANES, WLANES, cw)
    cw = lax.fori_loop(0, nw, win_body, 0)

    trips = jnp.where(wid == _NW - 1, 1, 0)
    for toff, tsize in tails:
        def tail_body(_, c, _toff=toff, _tsize=tsize):
            return do_window(jnp.int32(_toff), _tsize, c)
        cw = lax.fori_loop(0, trips, tail_body, cw)

    def drain_body(_, c):
        pltpu.make_async_copy(stage_hbm.at[pl.ds(0, 1)],
                              rowtmp_v.at[pl.ds(0, 1)], sem_w).wait()
        return c
    lax.fori_loop(0, jnp.minimum(cw, RING), drain_body, 0)


def _sc_filter_body(users_hbm, movies_hbm, utab_hbm, mtab_hbm,
                    u_out_hbm, m_out_hbm,
                    uidx_v, midx_v, mpos_v, mlane_v, wpos_v, wlane_v,
                    buf_v, rowtmp_v, sem_s, sem_w):
    wid = lax.axis_index("s") * _NC + lax.axis_index("c")
    pltpu.sync_copy(users_hbm, uidx_v)
    pltpu.sync_copy(movies_hbm, midx_v)
    _phase(utab_hbm, u_out_hbm, uidx_v, mpos_v, mlane_v, wpos_v, wlane_v,
           buf_v, rowtmp_v, sem_s, sem_w, wid, N_U, _U_TAIL)
    _phase(mtab_hbm, m_out_hbm, midx_v, mpos_v, mlane_v, wpos_v, wlane_v,
           buf_v, rowtmp_v, sem_s, sem_w, wid, N_M, _M_TAIL)


def _sc_filter(users, movies, utab_t, mtab_t):
    mesh = plsc.VectorSubcoreMesh(core_axis_name="c", subcore_axis_name="s")
    k = pl.kernel(
        _sc_filter_body,
        mesh=mesh,
        compiler_params=pltpu.CompilerParams(needs_layout_passes=False,
                                             disable_bounds_checks=True),
        out_type=[
            jax.ShapeDtypeStruct((BATCH, EMBD), jnp.float32),
            jax.ShapeDtypeStruct((BATCH, EMBD), jnp.float32),
        ],
        scratch_types=[
            pltpu.VMEM((BATCH,), jnp.int32),
            pltpu.VMEM((BATCH,), jnp.int32),
            pltpu.VMEM((BATCH + 16,), jnp.int32),
            pltpu.VMEM((BATCH + 16,), jnp.int32),
            pltpu.VMEM((BATCH + 16,), jnp.int32),
            pltpu.VMEM((BATCH + 16,), jnp.int32),
            pltpu.VMEM((EMBD, WLANES), jnp.float32),
            pltpu.VMEM((RING, EMBD), jnp.float32),
            pltpu.SemaphoreType.DMA,
            pltpu.SemaphoreType.DMA,
        ],
    )
    return k(users, movies, utab_t, mtab_t)


def _tc_head_body(u_ref, m_ref, wu_ref, wm_ref, b_ref, out_ref):
    acc = jnp.dot(u_ref[...], wu_ref[...], preferred_element_type=jnp.float32)
    acc += jnp.dot(m_ref[...], wm_ref[...], preferred_element_type=jnp.float32)
    acc += b_ref[...]
    mx = jnp.max(acc, axis=-1, keepdims=True)
    e = jnp.exp(acc - mx)
    s = jnp.sum(e, axis=-1, keepdims=True)
    out_ref[...] = e / s


def _tc_head(u_rows, m_rows, wu, wm, b8):
    blk = 2048
    grid = BATCH // blk
    return pl.pallas_call(
        _tc_head_body,
        grid=(grid,),
        in_specs=[
            pl.BlockSpec((blk, EMBD), lambda i: (i, 0)),
            pl.BlockSpec((blk, EMBD), lambda i: (i, 0)),
            pl.BlockSpec((EMBD, OUT_PAD), lambda i: (0, 0)),
            pl.BlockSpec((EMBD, OUT_PAD), lambda i: (0, 0)),
            pl.BlockSpec((1, OUT_PAD), lambda i: (0, 0)),
        ],
        out_specs=pl.BlockSpec((blk, OUT_PAD), lambda i: (i, 0)),
        out_shape=jax.ShapeDtypeStruct((BATCH, OUT_PAD), jnp.float32),
    )(u_rows, m_rows, wu, wm, b8)


def kernel(users, movies, user_table, movie_table, W, b):
    u_rows, m_rows = _sc_filter(users, movies, user_table.T, movie_table.T)
    wu = jnp.pad(W[:EMBD], ((0, 0), (0, OUT_PAD - 5)))
    wm = jnp.pad(W[EMBD:], ((0, 0), (0, OUT_PAD - 5)))
    b8 = jnp.pad(b, (0, OUT_PAD - 5), constant_values=-1e30).reshape(1, OUT_PAD)
    out8 = _tc_head(u_rows, m_rows, wu, wm, b8)
    return out8[:, :5]


# speedup vs baseline: 2.5222x; 1.3345x over previous
import functools

import jax
import jax.numpy as jnp
from jax import lax
from jax.experimental import pallas as pl
from jax.experimental.pallas import tpu as pltpu
from jax.experimental.pallas import tpu_sc as plsc

BATCH = 16384
EMBD = 64
OUT_PAD = 8
WLANES = 256
RING = 8

_info = plsc.get_sparse_core_info()
_NC = _info.num_cores
_NS = _info.num_subcores
_NW = _NC * _NS

N_U = 1000000
N_M = 100000
_U_TAIL = [(999936, 128)]
_M_TAIL = [(99840, 128), (99968, 128)]

_IOTA = None


def _phase(tab_hbm, stage_hbm, idx_hbm, idx_v, mpos_v, mlane_v, wpk_v,
           buf3_v, rowtmp_v, sem_s, sem_w, wid, n_total, tails):
    iota = lax.iota(jnp.int32, 16)
    shard = n_total // _NW
    lo = lax.bitwise_and(shard * wid, ~(WLANES - 1))
    hi_main = lax.bitwise_and(shard * (wid + 1), ~(WLANES - 1))
    hi = jnp.where(wid == _NW - 1, n_total, hi_main)

    pltpu.sync_copy(idx_hbm, idx_v)

    def scan_body(g, c):
        v = idx_v[pl.ds(g * 16, 16)]
        inr = jnp.logical_and(v >= lo, v < hi)
        plsc.store_compressed(mpos_v.at[pl.ds(c, 16)], iota + g * 16, mask=inr)
        plsc.store_compressed(mlane_v.at[pl.ds(c, 16)], v, mask=inr)
        return c + plsc.all_reduce_population_count(inr)[0]
    n = lax.fori_loop(0, BATCH // 16, scan_body, 0)

    nw = lax.shift_right_logical(hi_main - lo, 8)

    def start_w(w):
        wlo = pl.multiple_of(lo + w * WLANES, 128)
        par = lax.bitwise_and(w, 1)
        pltpu.async_copy(tab_hbm.at[:, pl.ds(wlo, WLANES)],
                         buf3_v.at[par], sem_s)

    def wait_w():
        pltpu.make_async_copy(tab_hbm.at[:, pl.ds(0, WLANES)],
                              buf3_v.at[0], sem_s).wait()

    def process_window(wlo, wsize, par, cw):
        def act_body(g, c2):
            mv = mlane_v[pl.ds(g * 16, 16)]
            pv = mpos_v[pl.ds(g * 16, 16)]
            a = jnp.logical_and(mv >= wlo, mv < wlo + wsize)
            a = jnp.logical_and(a, iota < n - g * 16)
            pk = lax.shift_left(pv, 9) + (mv - wlo)
            plsc.store_compressed(wpk_v.at[pl.ds(c2, 16)], pk, mask=a)
            return c2 + plsc.all_reduce_population_count(a)[0]
        na = lax.fori_loop(0, (n + 15) // 16, act_body, 0)

        parv = jnp.broadcast_to(par, (16,))

        def ext_body(i, c3):
            pk = wpk_v[pl.ds(i, 16)][0]
            pos = lax.shift_right_logical(pk, 9)
            llv = jnp.broadcast_to(lax.bitwise_and(pk, WLANES - 1), (16,))
            slot = lax.bitwise_and(c3, RING - 1)
            for g4 in range(4):
                col = plsc.load_gather(buf3_v, [parv, iota + g4 * 16, llv])
                rowtmp_v[slot, pl.ds(g4 * 16, 16)] = col

            @pl.when(c3 >= RING)
            def _():
                pltpu.make_async_copy(stage_hbm.at[pl.ds(0, 1)],
                                      rowtmp_v.at[pl.ds(0, 1)], sem_w).wait()
            pltpu.async_copy(rowtmp_v.at[pl.ds(slot, 1)],
                             stage_hbm.at[pl.ds(pos, 1)], sem_w)
            return c3 + 1
        return lax.fori_loop(0, na, ext_body, cw)

    start_w(0)

    def win_body(w, cw):
        wait_w()

        @pl.when(w + 1 < nw)
        def _():
            start_w(w + 1)
        return process_window(pl.multiple_of(lo + w * WLANES, 128), WLANES,
                              lax.bitwise_and(w, 1), cw)
    cw = lax.fori_loop(0, nw, win_body, 0)

    trips = jnp.where(wid == _NW - 1, 1, 0)
    for toff, tsize in tails:
        def tail_body(_, c, _toff=toff, _tsize=tsize):
            _toff = pl.multiple_of(jnp.int32(_toff), 128)
            pltpu.async_copy(tab_hbm.at[:, pl.ds(_toff, _tsize)],
                             buf3_v.at[0, :, pl.ds(0, _tsize)], sem_s).wait()
            return process_window(jnp.int32(_toff), _tsize, 0, c)
        cw = lax.fori_loop(0, trips, tail_body, cw)

    def drain_body(_, c):
        pltpu.make_async_copy(stage_hbm.at[pl.ds(0, 1)],
                              rowtmp_v.at[pl.ds(0, 1)], sem_w).wait()
        return c
    lax.fori_loop(0, jnp.minimum(cw, RING), drain_body, 0)


def _sc_filter_body(users_hbm, movies_hbm, utab_hbm, mtab_hbm,
                    u_out_hbm, m_out_hbm,
                    idx_v, mpos_v, mlane_v, wpk_v,
                    buf3_v, rowtmp_v, sem_s, sem_w):
    wid = lax.axis_index("s") * _NC + lax.axis_index("c")
    _phase(utab_hbm, u_out_hbm, users_hbm, idx_v, mpos_v, mlane_v, wpk_v,
           buf3_v, rowtmp_v, sem_s, sem_w, wid, N_U, _U_TAIL)
    _phase(mtab_hbm, m_out_hbm, movies_hbm, idx_v, mpos_v, mlane_v, wpk_v,
           buf3_v, rowtmp_v, sem_s, sem_w, wid, N_M, _M_TAIL)


def _sc_filter(users, movies, utab_t, mtab_t):
    mesh = plsc.VectorSubcoreMesh(core_axis_name="c", subcore_axis_name="s")
    k = pl.kernel(
        _sc_filter_body,
        mesh=mesh,
        compiler_params=pltpu.CompilerParams(needs_layout_passes=False,
                                             disable_bounds_checks=True),
        out_type=[
            jax.ShapeDtypeStruct((BATCH, EMBD), jnp.float32),
            jax.ShapeDtypeStruct((BATCH, EMBD), jnp.float32),
        ],
        scratch_types=[
            pltpu.VMEM((BATCH,), jnp.int32),
            pltpu.VMEM((BATCH + 16,), jnp.int32),
            pltpu.VMEM((BATCH + 16,), jnp.int32),
            pltpu.VMEM((BATCH + 16,), jnp.int32),
            pltpu.VMEM((2, EMBD, WLANES), jnp.float32),
            pltpu.VMEM((RING, EMBD), jnp.float32),
            pltpu.SemaphoreType.DMA,
            pltpu.SemaphoreType.DMA,
        ],
    )
    return k(users, movies, utab_t, mtab_t)


def _tc_head_body(u_ref, m_ref, wu_ref, wm_ref, b_ref, out_ref):
    acc = jnp.dot(u_ref[...], wu_ref[...], preferred_element_type=jnp.float32)
    acc += jnp.dot(m_ref[...], wm_ref[...], preferred_element_type=jnp.float32)
    acc += b_ref[...]
    mx = jnp.max(acc, axis=-1, keepdims=True)
    e = jnp.exp(acc - mx)
    s = jnp.sum(e, axis=-1, keepdims=True)
    out_ref[...] = e / s


def _tc_head(u_rows, m_rows, wu, wm, b8):
    blk = 2048
    grid = BATCH // blk
    return pl.pallas_call(
        _tc_head_body,
        grid=(grid,),
        in_specs=[
            pl.BlockSpec((blk, EMBD), lambda i: (i, 0)),
            pl.BlockSpec((blk, EMBD), lambda i: (i, 0)),
            pl.BlockSpec((EMBD, OUT_PAD), lambda i: (0, 0)),
            pl.BlockSpec((EMBD, OUT_PAD), lambda i: (0, 0)),
            pl.BlockSpec((1, OUT_PAD), lambda i: (0, 0)),
        ],
        out_specs=pl.BlockSpec((blk, OUT_PAD), lambda i: (i, 0)),
        out_shape=jax.ShapeDtypeStruct((BATCH, OUT_PAD), jnp.float32),
    )(u_rows, m_rows, wu, wm, b8)


def kernel(users, movies, user_table, movie_table, W, b):
    u_rows, m_rows = _sc_filter(users, movies, user_table.T, movie_table.T)
    wu = jnp.pad(W[:EMBD], ((0, 0), (0, OUT_PAD - 5)))
    wm = jnp.pad(W[EMBD:], ((0, 0), (0, OUT_PAD - 5)))
    b8 = jnp.pad(b, (0, OUT_PAD - 5), constant_values=-1e30).reshape(1, OUT_PAD)
    out8 = _tc_head(u_rows, m_rows, wu, wm, b8)
    return out8[:, :5]


# speedup vs baseline: 3.0057x; 1.1917x over previous
import functools

import jax
import jax.numpy as jnp
from jax import lax
from jax.experimental import pallas as pl
from jax.experimental.pallas import tpu as pltpu
from jax.experimental.pallas import tpu_sc as plsc

BATCH = 16384
EMBD = 64
OUT_PAD = 8
WLANES = 512
RING = 8

_info = plsc.get_sparse_core_info()
_NC = _info.num_cores
_NS = _info.num_subcores
_NW = _NC * _NS

N_U = 1000000
N_M = 100000
_U_TAIL = [(999936, 128)]
_M_TAIL = [(99840, 128), (99968, 128)]

_IOTA = None


def _phase(tab_hbm, stage_hbm, idx_hbm, idx_v, mpk_v, wpk_v,
           buf3_v, rowtmp_v, sem_s, sem_w, wid, n_total, tails):
    iota = lax.iota(jnp.int32, 16)
    shard = n_total // _NW
    lo = lax.bitwise_and(shard * wid, ~(WLANES - 1))
    hi_main = lax.bitwise_and(shard * (wid + 1), ~(WLANES - 1))
    hi = jnp.where(wid == _NW - 1, n_total, hi_main)

    pltpu.sync_copy(idx_hbm, idx_v)

    def scan_body(g, c):
        v = idx_v[pl.ds(g * 16, 16)]
        inr = jnp.logical_and(v >= lo, v < hi)
        pk = lax.shift_left(iota + g * 16, 15) + (v - lo)
        plsc.store_compressed(mpk_v.at[pl.ds(c, 16)], pk, mask=inr)
        return c + plsc.all_reduce_population_count(inr)[0]
    n = lax.fori_loop(0, BATCH // 16, scan_body, 0)

    nw = lax.shift_right_logical(hi_main - lo, 9)

    def start_w(w):
        wlo = pl.multiple_of(lo + w * WLANES, 128)
        par = lax.bitwise_and(w, 1)
        pltpu.async_copy(tab_hbm.at[:, pl.ds(wlo, WLANES)],
                         buf3_v.at[par], sem_s)

    def wait_w():
        pltpu.make_async_copy(tab_hbm.at[:, pl.ds(0, WLANES)],
                              buf3_v.at[0], sem_s).wait()

    def process_window(wloc, wsize, par, cw):
        def act_body(g, c2):
            pk = mpk_v[pl.ds(g * 16, 16)]
            lpk = lax.bitwise_and(pk, 32767)
            a = jnp.logical_and(lpk >= wloc, lpk < wloc + wsize)
            a = jnp.logical_and(a, iota < n - g * 16)
            plsc.store_compressed(wpk_v.at[pl.ds(c2, 16)], pk, mask=a)
            return c2 + plsc.all_reduce_population_count(a)[0]
        na = lax.fori_loop(0, (n + 15) // 16, act_body, 0)

        parv = jnp.broadcast_to(par, (16,))

        def ext_body(i, c3):
            pk = wpk_v[pl.ds(i, 16)][0]
            pos = lax.shift_right_logical(pk, 15)
            llv = jnp.broadcast_to(lax.bitwise_and(pk, 32767) - wloc, (16,))
            slot = lax.bitwise_and(c3, RING - 1)
            for g4 in range(4):
                col = plsc.load_gather(buf3_v, [parv, iota + g4 * 16, llv])
                rowtmp_v[slot, pl.ds(g4 * 16, 16)] = col

            @pl.when(c3 >= RING)
            def _():
                pltpu.make_async_copy(stage_hbm.at[pl.ds(0, 1)],
                                      rowtmp_v.at[pl.ds(0, 1)], sem_w).wait()
            pltpu.async_copy(rowtmp_v.at[pl.ds(slot, 1)],
                             stage_hbm.at[pl.ds(pos, 1)], sem_w)
            return c3 + 1
        return lax.fori_loop(0, na, ext_body, cw)

    start_w(0)

    def win_body(w, cw):
        wait_w()

        @pl.when(w + 1 < nw)
        def _():
            start_w(w + 1)
        return process_window(w * WLANES, WLANES, lax.bitwise_and(w, 1), cw)
    cw = lax.fori_loop(0, nw, win_body, 0)

    trips = jnp.where(wid == _NW - 1, 1, 0)
    for toff, tsize in tails:
        def tail_body(_, c, _toff=toff, _tsize=tsize):
            _aoff = pl.multiple_of(jnp.int32(_toff), 128)
            pltpu.async_copy(tab_hbm.at[:, pl.ds(_aoff, _tsize)],
                             buf3_v.at[0, :, pl.ds(0, _tsize)], sem_s).wait()
            return process_window(jnp.int32(_toff) - lo, _tsize, 0, c)
        cw = lax.fori_loop(0, trips, tail_body, cw)

    def drain_body(_, c):
        pltpu.make_async_copy(stage_hbm.at[pl.ds(0, 1)],
                              rowtmp_v.at[pl.ds(0, 1)], sem_w).wait()
        return c
    lax.fori_loop(0, jnp.minimum(cw, RING), drain_body, 0)


def _sc_filter_body(users_hbm, movies_hbm, utab_hbm, mtab_hbm,
                    u_out_hbm, m_out_hbm,
                    idx_v, mpk_v, wpk_v,
                    buf3_v, rowtmp_v, sem_s, sem_w):
    wid = lax.axis_index("s") * _NC + lax.axis_index("c")
    _phase(utab_hbm, u_out_hbm, users_hbm, idx_v, mpk_v, wpk_v,
           buf3_v, rowtmp_v, sem_s, sem_w, wid, N_U, _U_TAIL)
    _phase(mtab_hbm, m_out_hbm, movies_hbm, idx_v, mpk_v, wpk_v,
           buf3_v, rowtmp_v, sem_s, sem_w, wid, N_M, _M_TAIL)


def _sc_filter(users, movies, utab_t, mtab_t):
    mesh = plsc.VectorSubcoreMesh(core_axis_name="c", subcore_axis_name="s")
    k = pl.kernel(
        _sc_filter_body,
        mesh=mesh,
        compiler_params=pltpu.CompilerParams(needs_layout_passes=False,
                                             disable_bounds_checks=True),
        out_type=[
            jax.ShapeDtypeStruct((BATCH, EMBD), jnp.float32),
            jax.ShapeDtypeStruct((BATCH, EMBD), jnp.float32),
        ],
        scratch_types=[
            pltpu.VMEM((BATCH,), jnp.int32),
            pltpu.VMEM((BATCH + 16,), jnp.int32),
            pltpu.VMEM((BATCH + 16,), jnp.int32),
            pltpu.VMEM((2, EMBD, WLANES), jnp.float32),
            pltpu.VMEM((RING, EMBD), jnp.float32),
            pltpu.SemaphoreType.DMA,
            pltpu.SemaphoreType.DMA,
        ],
    )
    return k(users, movies, utab_t, mtab_t)


def _tc_head_body(u_ref, m_ref, wu_ref, wm_ref, b_ref, out_ref):
    acc = jnp.dot(u_ref[...], wu_ref[...], preferred_element_type=jnp.float32)
    acc += jnp.dot(m_ref[...], wm_ref[...], preferred_element_type=jnp.float32)
    acc += b_ref[...]
    mx = jnp.max(acc, axis=-1, keepdims=True)
    e = jnp.exp(acc - mx)
    s = jnp.sum(e, axis=-1, keepdims=True)
    out_ref[...] = e / s


def _tc_head(u_rows, m_rows, wu, wm, b8):
    blk = 2048
    grid = BATCH // blk
    return pl.pallas_call(
        _tc_head_body,
        grid=(grid,),
        in_specs=[
            pl.BlockSpec((blk, EMBD), lambda i: (i, 0)),
            pl.BlockSpec((blk, EMBD), lambda i: (i, 0)),
            pl.BlockSpec((EMBD, OUT_PAD), lambda i: (0, 0)),
            pl.BlockSpec((EMBD, OUT_PAD), lambda i: (0, 0)),
            pl.BlockSpec((1, OUT_PAD), lambda i: (0, 0)),
        ],
        out_specs=pl.BlockSpec((blk, OUT_PAD), lambda i: (i, 0)),
        out_shape=jax.ShapeDtypeStruct((BATCH, OUT_PAD), jnp.float32),
    )(u_rows, m_rows, wu, wm, b8)


def kernel(users, movies, user_table, movie_table, W, b):
    u_rows, m_rows = _sc_filter(users, movies, user_table.T, movie_table.T)
    wu = jnp.pad(W[:EMBD], ((0, 0), (0, OUT_PAD - 5)))
    wm = jnp.pad(W[EMBD:], ((0, 0), (0, OUT_PAD - 5)))
    b8 = jnp.pad(b, (0, OUT_PAD - 5), constant_values=-1e30).reshape(1, OUT_PAD)
    out8 = _tc_head(u_rows, m_rows, wu, wm, b8)
    return out8[:, :5]


# speedup vs baseline: 3.5204x; 1.1712x over previous
import functools

import jax
import jax.numpy as jnp
from jax import lax
from jax.experimental import pallas as pl
from jax.experimental.pallas import tpu as pltpu
from jax.experimental.pallas import tpu_sc as plsc

BATCH = 16384
EMBD = 64
OUT_PAD = 8
WLANES = 256
NBUF = 4
RING = 8

_info = plsc.get_sparse_core_info()
_NC = _info.num_cores
_NS = _info.num_subcores
_NW = _NC * _NS

N_U = 1000000
N_M = 100000
_U_TAIL = [(999936, 128)]
_M_TAIL = [(99840, 128), (99968, 128)]

_IOTA = None


def _phase(tab_hbm, stage_hbm, idx_hbm, idx_v, mpk_v, wpk_v,
           buf3_v, rowtmp_v, sem_s, sem_w, wid, n_total, tails):
    iota = lax.iota(jnp.int32, 16)
    shard = n_total // _NW
    lo = lax.bitwise_and(shard * wid, ~(WLANES - 1))
    hi_main = lax.bitwise_and(shard * (wid + 1), ~(WLANES - 1))
    hi = jnp.where(wid == _NW - 1, n_total, hi_main)

    nw = lax.shift_right_logical(hi_main - lo, 8)

    def start_w(w):
        wlo = pl.multiple_of(lo + w * WLANES, 128)
        par = lax.bitwise_and(w, NBUF - 1)
        pltpu.async_copy(tab_hbm.at[:, pl.ds(wlo, WLANES)],
                         buf3_v.at[par], sem_s)

    def wait_w():
        pltpu.make_async_copy(tab_hbm.at[:, pl.ds(0, WLANES)],
                              buf3_v.at[0], sem_s).wait()

    start_w(0)
    start_w(1)
    start_w(2)

    pltpu.sync_copy(idx_hbm, idx_v)

    def scan_body(g, c):
        v = idx_v[pl.ds(g * 16, 16)]
        inr = jnp.logical_and(v >= lo, v < hi)
        pk = lax.shift_left(iota + g * 16, 15) + (v - lo)
        plsc.store_compressed(mpk_v.at[pl.ds(c, 16)], pk, mask=inr)
        return c + plsc.all_reduce_population_count(inr)[0]
    n = lax.fori_loop(0, BATCH // 16, scan_body, 0)

    def process_window(wloc, wsize, par, cw):
        def act_body(g, c2):
            pk = mpk_v[pl.ds(g * 16, 16)]
            lpk = lax.bitwise_and(pk, 32767)
            a = jnp.logical_and(lpk >= wloc, lpk < wloc + wsize)
            a = jnp.logical_and(a, iota < n - g * 16)
            plsc.store_compressed(wpk_v.at[pl.ds(c2, 16)], pk, mask=a)
            return c2 + plsc.all_reduce_population_count(a)[0]
        na = lax.fori_loop(0, (n + 15) // 16, act_body, 0)

        parv = jnp.broadcast_to(par, (16,))

        def ext_body(i, c3):
            pk = wpk_v[pl.ds(i, 16)][0]
            pos = lax.shift_right_logical(pk, 15)
            llv = jnp.broadcast_to(lax.bitwise_and(pk, 32767) - wloc, (16,))
            slot = lax.bitwise_and(c3, RING - 1)
            for g4 in range(4):
                col = plsc.load_gather(buf3_v, [parv, iota + g4 * 16, llv])
                rowtmp_v[slot, pl.ds(g4 * 16, 16)] = col

            @pl.when(c3 >= RING)
            def _():
                pltpu.make_async_copy(stage_hbm.at[pl.ds(0, 1)],
                                      rowtmp_v.at[pl.ds(0, 1)], sem_w).wait()
            pltpu.async_copy(rowtmp_v.at[pl.ds(slot, 1)],
                             stage_hbm.at[pl.ds(pos, 1)], sem_w)
            return c3 + 1
        return lax.fori_loop(0, na, ext_body, cw)

    def win_body(w, cw):
        wait_w()

        @pl.when(w + 3 < nw)
        def _():
            start_w(w + 3)
        return process_window(w * WLANES, WLANES,
                              lax.bitwise_and(w, NBUF - 1), cw)
    cw = lax.fori_loop(0, nw, win_body, 0)

    trips = jnp.where(wid == _NW - 1, 1, 0)
    for toff, tsize in tails:
        def tail_body(_, c, _toff=toff, _tsize=tsize):
            _aoff = pl.multiple_of(jnp.int32(_toff), 128)
            pltpu.async_copy(tab_hbm.at[:, pl.ds(_aoff, _tsize)],
                             buf3_v.at[0, :, pl.ds(0, _tsize)], sem_s).wait()
            return process_window(jnp.int32(_toff) - lo, _tsize, 0, c)
        cw = lax.fori_loop(0, trips, tail_body, cw)

    def drain_body(_, c):
        pltpu.make_async_copy(stage_hbm.at[pl.ds(0, 1)],
                              rowtmp_v.at[pl.ds(0, 1)], sem_w).wait()
        return c
    lax.fori_loop(0, jnp.minimum(cw, RING), drain_body, 0)


def _sc_filter_body(users_hbm, movies_hbm, utab_hbm, mtab_hbm,
                    u_out_hbm, m_out_hbm,
                    idx_v, mpk_v, wpk_v,
                    buf3_v, rowtmp_v, sem_s, sem_w):
    wid = lax.axis_index("s") * _NC + lax.axis_index("c")
    _phase(utab_hbm, u_out_hbm, users_hbm, idx_v, mpk_v, wpk_v,
           buf3_v, rowtmp_v, sem_s, sem_w, wid, N_U, _U_TAIL)
    _phase(mtab_hbm, m_out_hbm, movies_hbm, idx_v, mpk_v, wpk_v,
           buf3_v, rowtmp_v, sem_s, sem_w, wid, N_M, _M_TAIL)


def _sc_filter(users, movies, utab_t, mtab_t):
    mesh = plsc.VectorSubcoreMesh(core_axis_name="c", subcore_axis_name="s")
    k = pl.kernel(
        _sc_filter_body,
        mesh=mesh,
        compiler_params=pltpu.CompilerParams(needs_layout_passes=False,
                                             disable_bounds_checks=True),
        out_type=[
            jax.ShapeDtypeStruct((BATCH, EMBD), jnp.float32),
            jax.ShapeDtypeStruct((BATCH, EMBD), jnp.float32),
        ],
        scratch_types=[
            pltpu.VMEM((BATCH,), jnp.int32),
            pltpu.VMEM((BATCH + 16,), jnp.int32),
            pltpu.VMEM((BATCH + 16,), jnp.int32),
            pltpu.VMEM((NBUF, EMBD, WLANES), jnp.float32),
            pltpu.VMEM((RING, EMBD), jnp.float32),
            pltpu.SemaphoreType.DMA,
            pltpu.SemaphoreType.DMA,
        ],
    )
    return k(users, movies, utab_t, mtab_t)


def _tc_head_body(u_ref, m_ref, wu_ref, wm_ref, b_ref, out_ref):
    acc = jnp.dot(u_ref[...], wu_ref[...], preferred_element_type=jnp.float32)
    acc += jnp.dot(m_ref[...], wm_ref[...], preferred_element_type=jnp.float32)
    acc += b_ref[...]
    mx = jnp.max(acc, axis=-1, keepdims=True)
    e = jnp.exp(acc - mx)
    s = jnp.sum(e, axis=-1, keepdims=True)
    out_ref[...] = e / s


def _tc_head(u_rows, m_rows, wu, wm, b8):
    blk = 2048
    grid = BATCH // blk
    return pl.pallas_call(
        _tc_head_body,
        grid=(grid,),
        in_specs=[
            pl.BlockSpec((blk, EMBD), lambda i: (i, 0)),
            pl.BlockSpec((blk, EMBD), lambda i: (i, 0)),
            pl.BlockSpec((EMBD, OUT_PAD), lambda i: (0, 0)),
            pl.BlockSpec((EMBD, OUT_PAD), lambda i: (0, 0)),
            pl.BlockSpec((1, OUT_PAD), lambda i: (0, 0)),
        ],
        out_specs=pl.BlockSpec((blk, OUT_PAD), lambda i: (i, 0)),
        out_shape=jax.ShapeDtypeStruct((BATCH, OUT_PAD), jnp.float32),
    )(u_rows, m_rows, wu, wm, b8)


def kernel(users, movies, user_table, movie_table, W, b):
    u_rows, m_rows = _sc_filter(users, movies, user_table.T, movie_table.T)
    wu = jnp.pad(W[:EMBD], ((0, 0), (0, OUT_PAD - 5)))
    wm = jnp.pad(W[EMBD:], ((0, 0), (0, OUT_PAD - 5)))
    b8 = jnp.pad(b, (0, OUT_PAD - 5), constant_values=-1e30).reshape(1, OUT_PAD)
    out8 = _tc_head(u_rows, m_rows, wu, wm, b8)
    return out8[:, :5]
